# baseline (speedup 1.0000x reference)
import functools

import jax
import jax.numpy as jnp
from jax import lax
from jax.experimental import pallas as pl
from jax.experimental.pallas import tpu as pltpu
from jax.experimental.pallas import tpu_sc as plsc

_IN_CH = 128
_OUT_CH = 128
_BATCH = 4096
_W_COLS = _IN_CH * _OUT_CH
_ROW = _W_COLS + _OUT_CH

_NC = 2
_NS = 16
_NW = _NC * _NS
_BPW = _BATCH // _NW
_R = 4
_NGROUP = _BPW // _R


@jax.jit
def _lookup(table, idx, idx2):
    mesh = plsc.VectorSubcoreMesh(core_axis_name="c", subcore_axis_name="s")

    @functools.partial(
        pl.kernel,
        mesh=mesh,
        out_type=(
            jax.ShapeDtypeStruct((_BATCH, _W_COLS), jnp.float32),
            jax.ShapeDtypeStruct((_BATCH, _OUT_CH), jnp.float32),
        ),
        scratch_types=[
            pltpu.VMEM((_BPW,), jnp.int32),
            pltpu.VMEM((_BPW, 1), jnp.int32),
            pltpu.VMEM((_R, 1, _W_COLS), jnp.float32),
            pltpu.VMEM((_BPW, _OUT_CH), jnp.float32),
            pltpu.SemaphoreType.DMA,
            pltpu.SemaphoreType.DMA,
            pltpu.SemaphoreType.DMA,
            pltpu.SemaphoreType.DMA,
            pltpu.SemaphoreType.DMA,
            pltpu.SemaphoreType.DMA,
            pltpu.SemaphoreType.DMA,
            pltpu.SemaphoreType.DMA,
            pltpu.SemaphoreType.DMA,
        ],
    )
    def k(table_hbm, idx_hbm, idx2_hbm, w_hbm, b_hbm,
          idx_v, idx2_v, rows_v, bias_v,
          sb, sg0, sg1, sg2, sg3, so0, so1, so2, so3):
        sg = (sg0, sg1, sg2, sg3)
        so = (so0, so1, so2, so3)
        wid = lax.axis_index("s") * _NC + lax.axis_index("c")
        base = wid * _BPW
        pltpu.sync_copy(idx_hbm.at[wid], idx_v)
        pltpu.sync_copy(idx2_hbm.at[wid], idx2_v)

        bias_cp = pltpu.async_copy(
            table_hbm.at[idx_v, pl.ds(_W_COLS, _OUT_CH)], bias_v, sb)

        def gather_row(step, slot):
            pltpu.async_copy(
                table_hbm.at[idx2_v.at[step], pl.ds(0, _W_COLS)],
                rows_v.at[slot], sg[slot])

        for b in range(_R):
            gather_row(b, b)

        def body(g, carry):
            outs = []
            for b in range(_R):
                step = g * _R + b
                pltpu.make_async_copy(
                    table_hbm.at[idx2_v.at[step], pl.ds(0, _W_COLS)],
                    rows_v.at[b], sg[b]).wait()
                outs.append(pltpu.async_copy(
                    rows_v.at[b], w_hbm.at[pl.ds(base + step, 1), :], so[b]))
            for b in range(_R):
                outs[b].wait()

                @pl.when(g < _NGROUP - 1)
                def _():
                    gather_row((g + 1) * _R + b, b)

            return carry

        lax.fori_loop(0, _NGROUP, body, 0)

        bias_cp.wait()
        pltpu.sync_copy(bias_v, b_hbm.at[pl.ds(base, _BPW), :])

    return k(table, idx, idx2)


def kernel(table, obj_idx):
    idx = obj_idx.astype(jnp.int32).reshape(_NW, _BPW)
    w_flat, b_flat = _lookup(table, idx, idx.reshape(_NW, _BPW, 1))
    return (w_flat.reshape(_BATCH, _OUT_CH, _IN_CH),
            b_flat.reshape(_BATCH, 1, _OUT_CH))
